# Initial kernel scaffold; baseline (speedup 1.0000x reference)
import functools

import jax
import jax.numpy as jnp
from jax import lax
from jax.experimental import pallas as pl
from jax.experimental.pallas import tpu as pltpu
from jax.experimental.pallas import tpu_sc as plsc

NSEG = 1024
LANES = 16
NCORES = 2
NSUB = 16
CPP = 6
NPASS = 2


@functools.lru_cache(maxsize=None)
def _build(nbatch, nchan, npix, chunk):
    assert npix % chunk == 0 and chunk % LANES == 0
    nworkers = NCORES * NSUB
    groups = nworkers // nbatch
    ch_per_group = nchan // groups
    assert ch_per_group == CPP * NPASS
    nchunks = npix // chunk
    vregs = chunk // LANES

    mesh = plsc.VectorSubcoreMesh(
        core_axis_name="c", subcore_axis_name="s",
        num_cores=NCORES, num_subcores=NSUB)

    def body(img, spx, out, lab, val, *accs):
        cid = lax.axis_index("c")
        sid = lax.axis_index("s")
        wid = sid * NCORES + cid
        b = wid // groups
        grp = wid % groups
        ch_base = grp * ch_per_group

        lane = lax.iota(jnp.int32, LANES)
        neg = jnp.full((LANES,), -jnp.inf, dtype=jnp.float32)

        for p in range(NPASS):
            ch0 = ch_base + p * CPP

            def init_body(j, carry):
                for a in accs:
                    for r in range(LANES):
                        a[r, pl.ds(j * LANES, LANES)] = neg
                return carry
            lax.fori_loop(0, NSEG // LANES, init_body, 0)

            def chunk_body(t, carry):
                off = t * chunk
                pltpu.sync_copy(spx.at[b, pl.ds(off, chunk)], lab)
                pltpu.sync_copy(
                    img.at[b, pl.ds(ch0, CPP), pl.ds(off, chunk)], val)

                def inner(i, c2):
                    lb = lab[pl.ds(i * LANES, LANES)]
                    for c in range(CPP):
                        v = val[c, pl.ds(i * LANES, LANES)]
                        cur = plsc.load_gather(accs[c], [lane, lb])
                        plsc.store_scatter(
                            accs[c], [lane, lb], jnp.maximum(cur, v))
                    return c2
                lax.fori_loop(0, vregs, inner, 0)
                return carry
            lax.fori_loop(0, nchunks, chunk_body, 0)

            for c in range(CPP):
                a = accs[c]
                for step in (8, 4, 2, 1):
                    def red_body(j, carry, a=a, step=step):
                        for r in range(step):
                            x = a[r, pl.ds(j * LANES, LANES)]
                            y = a[r + step, pl.ds(j * LANES, LANES)]
                            a[r, pl.ds(j * LANES, LANES)] = jnp.maximum(x, y)
                        return carry
                    lax.fori_loop(0, NSEG // LANES, red_body, 0)
                pltpu.sync_copy(a.at[0], out.at[b, ch0 + c])

    run = pl.kernel(
        body,
        out_type=jax.ShapeDtypeStruct((nbatch, nchan, NSEG), jnp.float32),
        mesh=mesh,
        scratch_types=[
            pltpu.VMEM((chunk,), jnp.int32),
            pltpu.VMEM((CPP, chunk), jnp.float32),
        ] + [pltpu.VMEM((LANES, NSEG), jnp.float32)] * CPP,
    )
    return run


def kernel(img, spx):
    B, C, H, W = img.shape
    imgf = img.reshape(B, C, H * W)
    spxf = spx.reshape(B, H * W).astype(jnp.int32)
    run = _build(B, C, H * W, 2048)
    return run(imgf, spxf)


# speedup vs baseline: 2.0016x; 2.0016x over previous
import functools

import jax
import jax.numpy as jnp
from jax import lax
from jax.experimental import pallas as pl
from jax.experimental.pallas import tpu as pltpu
from jax.experimental.pallas import tpu_sc as plsc

NSEG = 1024
LANES = 16
NCORES = 2
NSUB = 16
CPP = 6
NPASS = 2


@functools.lru_cache(maxsize=None)
def _build(nbatch, nchan, npix, chunk):
    assert npix % chunk == 0 and chunk % LANES == 0
    nworkers = NCORES * NSUB
    groups = nworkers // nbatch
    ch_per_group = nchan // groups
    assert ch_per_group == CPP * NPASS
    nchunks = npix // chunk
    vregs = chunk // LANES

    mesh = plsc.VectorSubcoreMesh(
        core_axis_name="c", subcore_axis_name="s",
        num_cores=NCORES, num_subcores=NSUB)

    def body(img, spx, out, lab, val, *accs):
        cid = lax.axis_index("c")
        sid = lax.axis_index("s")
        wid = sid * NCORES + cid
        b = wid // groups
        grp = wid % groups
        ch_base = grp * ch_per_group

        lane = lax.iota(jnp.int32, LANES)
        neg = jnp.full((LANES,), -jnp.inf, dtype=jnp.float32)

        for p in range(NPASS):
            ch0 = ch_base + p * CPP

            def init_body(j, carry):
                for a in accs:
                    for r in range(LANES):
                        a[r, pl.ds(j * LANES, LANES)] = neg
                return carry
            lax.fori_loop(0, NSEG // LANES, init_body, 0)

            def chunk_body(t, carry):
                off = t * chunk
                pltpu.sync_copy(spx.at[b, pl.ds(off, chunk)], lab)
                pltpu.sync_copy(
                    img.at[b, pl.ds(ch0, CPP), pl.ds(off, chunk)], val)

                def inner(i, c2):
                    lb = lab[pl.ds(i * LANES, LANES)]
                    for c in range(CPP):
                        v = val[c, pl.ds(i * LANES, LANES)]
                        cur = plsc.load_gather(accs[c], [lane, lb])
                        plsc.store_scatter(
                            accs[c], [lane, lb], jnp.maximum(cur, v))
                    return c2
                lax.fori_loop(0, vregs, inner, 0)
                return carry
            lax.fori_loop(0, nchunks, chunk_body, 0)

            for c in range(CPP):
                a = accs[c]
                for step in (8, 4, 2, 1):
                    def red_body(j, carry, a=a, step=step):
                        for r in range(step):
                            x = a[r, pl.ds(j * LANES, LANES)]
                            y = a[r + step, pl.ds(j * LANES, LANES)]
                            a[r, pl.ds(j * LANES, LANES)] = jnp.maximum(x, y)
                        return carry
                    lax.fori_loop(0, NSEG // LANES, red_body, 0)
                pltpu.sync_copy(a.at[0], out.at[b, ch0 + c])

    run = pl.kernel(
        body,
        out_type=jax.ShapeDtypeStruct((nbatch, nchan, NSEG), jnp.float32),
        mesh=mesh,
        compiler_params=pltpu.CompilerParams(
            use_tc_tiling_on_sc=False, needs_layout_passes=False),
        scratch_types=[
            pltpu.VMEM((chunk,), jnp.int32),
            pltpu.VMEM((CPP, chunk), jnp.float32),
        ] + [pltpu.VMEM((LANES, NSEG), jnp.float32)] * CPP,
    )
    return run


def kernel(img, spx):
    B, C, H, W = img.shape
    imgf = img.reshape(B, C, H * W)
    spxf = spx.reshape(B, H * W).astype(jnp.int32)
    run = _build(B, C, H * W, 2048)
    return run(imgf, spxf)


# speedup vs baseline: 2.7651x; 1.3814x over previous
import functools

import jax
import jax.numpy as jnp
from jax import lax
from jax.experimental import pallas as pl
from jax.experimental.pallas import tpu as pltpu
from jax.experimental.pallas import tpu_sc as plsc

NSEG = 1024
LANES = 16
NCORES = 2
NSUB = 16
CPP = 6
NPASS = 2


@functools.lru_cache(maxsize=None)
def _build(nbatch, nchan, npix, chunk):
    assert npix % chunk == 0 and chunk % LANES == 0
    nworkers = NCORES * NSUB
    groups = nworkers // nbatch
    ch_per_group = nchan // groups
    assert ch_per_group == CPP * NPASS
    nchunks = npix // chunk
    vregs = chunk // LANES

    mesh = plsc.VectorSubcoreMesh(
        core_axis_name="c", subcore_axis_name="s",
        num_cores=NCORES, num_subcores=NSUB)

    def body(img, spx, out, lab, val, *accs):
        cid = lax.axis_index("c")
        sid = lax.axis_index("s")
        wid = sid * NCORES + cid
        b = wid // groups
        grp = wid % groups
        ch_base = grp * ch_per_group

        lane = lax.iota(jnp.int32, LANES)
        neg = jnp.full((LANES,), -jnp.inf, dtype=jnp.float32)

        for p in range(NPASS):
            ch0 = ch_base + p * CPP

            def init_body(j, carry):
                for a in accs:
                    for r in range(LANES):
                        a[r, pl.ds(j * LANES, LANES)] = neg
                return carry
            lax.fori_loop(0, NSEG // LANES, init_body, 0)

            def chunk_body(t, carry):
                off = t * chunk
                pltpu.sync_copy(spx.at[b, pl.ds(off, chunk)], lab)
                pltpu.sync_copy(
                    img.at[b, pl.ds(ch0, CPP), pl.ds(off, chunk)], val)

                def inner(i, c2):
                    lb = lab[pl.ds(i * LANES, LANES)]
                    vs = [val[c, pl.ds(i * LANES, LANES)]
                          for c in range(CPP)]
                    curs = [plsc.load_gather(accs[c], [lane, lb])
                            for c in range(CPP)]
                    news = [jnp.maximum(curs[c], vs[c]) for c in range(CPP)]
                    for c in range(CPP):
                        plsc.store_scatter(accs[c], [lane, lb], news[c])
                    return c2
                lax.fori_loop(0, vregs, inner, 0)
                return carry
            lax.fori_loop(0, nchunks, chunk_body, 0)

            for c in range(CPP):
                a = accs[c]
                for step in (8, 4, 2, 1):
                    def red_body(j, carry, a=a, step=step):
                        for r in range(step):
                            x = a[r, pl.ds(j * LANES, LANES)]
                            y = a[r + step, pl.ds(j * LANES, LANES)]
                            a[r, pl.ds(j * LANES, LANES)] = jnp.maximum(x, y)
                        return carry
                    lax.fori_loop(0, NSEG // LANES, red_body, 0)
                pltpu.sync_copy(a.at[0], out.at[b, ch0 + c])

    run = pl.kernel(
        body,
        out_type=jax.ShapeDtypeStruct((nbatch, nchan, NSEG), jnp.float32),
        mesh=mesh,
        compiler_params=pltpu.CompilerParams(
            use_tc_tiling_on_sc=False, needs_layout_passes=False),
        scratch_types=[
            pltpu.VMEM((chunk,), jnp.int32),
            pltpu.VMEM((CPP, chunk), jnp.float32),
        ] + [pltpu.VMEM((LANES, NSEG), jnp.float32)] * CPP,
    )
    return run


def kernel(img, spx):
    B, C, H, W = img.shape
    imgf = img.reshape(B, C, H * W)
    spxf = spx.reshape(B, H * W).astype(jnp.int32)
    run = _build(B, C, H * W, 2048)
    return run(imgf, spxf)


# speedup vs baseline: 3.8619x; 1.3966x over previous
import functools

import jax
import jax.numpy as jnp
from jax import lax
from jax.experimental import pallas as pl
from jax.experimental.pallas import tpu as pltpu
from jax.experimental.pallas import tpu_sc as plsc

NSEG = 1024
LANES = 16
NCORES = 2
NSUB = 16
CPP = 6
NPASS = 2


@functools.lru_cache(maxsize=None)
def _build(nbatch, nchan, npix, chunk):
    assert npix % chunk == 0 and chunk % LANES == 0
    nworkers = NCORES * NSUB
    groups = nworkers // nbatch
    ch_per_group = nchan // groups
    assert ch_per_group == CPP * NPASS
    nchunks = npix // chunk
    vregs = chunk // LANES

    mesh = plsc.VectorSubcoreMesh(
        core_axis_name="c", subcore_axis_name="s",
        num_cores=NCORES, num_subcores=NSUB)

    def body(img, spx, out, lab0, val0, lab1, val1, sem0, sem1, *accs):
        cid = lax.axis_index("c")
        sid = lax.axis_index("s")
        wid = sid * NCORES + cid
        b = wid // groups
        grp = wid % groups
        ch_base = grp * ch_per_group

        lane = lax.iota(jnp.int32, LANES)
        neg = jnp.full((LANES,), -jnp.inf, dtype=jnp.float32)

        for p in range(NPASS):
            ch0 = ch_base + p * CPP

            def init_body(j, carry):
                for a in accs:
                    for r in range(LANES):
                        a[r, pl.ds(j * LANES, LANES)] = neg
                return carry
            lax.fori_loop(0, NSEG // LANES, init_body, 0)

            def copies(t, lb_buf, vl_buf, sem):
                off = t * chunk
                return (
                    pltpu.make_async_copy(
                        spx.at[b, pl.ds(off, chunk)], lb_buf, sem),
                    pltpu.make_async_copy(
                        img.at[b, pl.ds(ch0, CPP), pl.ds(off, chunk)],
                        vl_buf, sem),
                )

            def start(t, lb_buf, vl_buf, sem):
                for d in copies(t, lb_buf, vl_buf, sem):
                    d.start()

            def wait(t, lb_buf, vl_buf, sem):
                for d in copies(t, lb_buf, vl_buf, sem):
                    d.wait()

            def compute(lab, val):
                def inner(i, c2):
                    lb = lab[pl.ds(i * LANES, LANES)]
                    vs = [val[c, pl.ds(i * LANES, LANES)]
                          for c in range(CPP)]
                    curs = [plsc.load_gather(accs[c], [lane, lb])
                            for c in range(CPP)]
                    news = [jnp.maximum(curs[c], vs[c]) for c in range(CPP)]
                    for c in range(CPP):
                        plsc.store_scatter(accs[c], [lane, lb], news[c])
                    return c2
                lax.fori_loop(0, vregs, inner, 0)

            start(0, lab0, val0, sem0)

            def chunk_body(u, carry):
                t0 = 2 * u
                wait(t0, lab0, val0, sem0)
                start(t0 + 1, lab1, val1, sem1)
                compute(lab0, val0)
                wait(t0 + 1, lab1, val1, sem1)

                @pl.when(u + 1 < nchunks // 2)
                def _():
                    start(t0 + 2, lab0, val0, sem0)
                compute(lab1, val1)
                return carry
            lax.fori_loop(0, nchunks // 2, chunk_body, 0)

            for c in range(CPP):
                a = accs[c]
                for step in (8, 4, 2, 1):
                    def red_body(j, carry, a=a, step=step):
                        for r in range(step):
                            x = a[r, pl.ds(j * LANES, LANES)]
                            y = a[r + step, pl.ds(j * LANES, LANES)]
                            a[r, pl.ds(j * LANES, LANES)] = jnp.maximum(x, y)
                        return carry
                    lax.fori_loop(0, NSEG // LANES, red_body, 0)
                pltpu.sync_copy(a.at[0], out.at[b, ch0 + c])

    run = pl.kernel(
        body,
        out_type=jax.ShapeDtypeStruct((nbatch, nchan, NSEG), jnp.float32),
        mesh=mesh,
        compiler_params=pltpu.CompilerParams(
            use_tc_tiling_on_sc=False, needs_layout_passes=False),
        scratch_types=[
            pltpu.VMEM((chunk,), jnp.int32),
            pltpu.VMEM((CPP, chunk), jnp.float32),
            pltpu.VMEM((chunk,), jnp.int32),
            pltpu.VMEM((CPP, chunk), jnp.float32),
            pltpu.SemaphoreType.DMA,
            pltpu.SemaphoreType.DMA,
        ] + [pltpu.VMEM((LANES, NSEG), jnp.float32)] * CPP,
    )
    return run


def kernel(img, spx):
    B, C, H, W = img.shape
    imgf = img.reshape(B, C, H * W)
    spxf = spx.reshape(B, H * W).astype(jnp.int32)
    run = _build(B, C, H * W, 2048)
    return run(imgf, spxf)


# speedup vs baseline: 4.2215x; 1.0931x over previous
import functools

import jax
import jax.numpy as jnp
from jax import lax
from jax.experimental import pallas as pl
from jax.experimental.pallas import tpu as pltpu
from jax.experimental.pallas import tpu_sc as plsc

NSEG = 1024
LANES = 16
NCORES = 2
NSUB = 16
CPP = 6
NPASS = 2


@functools.lru_cache(maxsize=None)
def _build(nbatch, nchan, npix, chunk):
    assert npix % chunk == 0 and chunk % LANES == 0
    nworkers = NCORES * NSUB
    groups = nworkers // nbatch
    ch_per_group = nchan // groups
    assert ch_per_group == CPP * NPASS
    nchunks = npix // chunk
    vregs = chunk // LANES

    mesh = plsc.VectorSubcoreMesh(
        core_axis_name="c", subcore_axis_name="s",
        num_cores=NCORES, num_subcores=NSUB)

    def body(img, spx, out, lab0, val0, lab1, val1, sem0, sem1, *accs):
        cid = lax.axis_index("c")
        sid = lax.axis_index("s")
        wid = sid * NCORES + cid
        b = wid // groups
        grp = wid % groups
        ch_base = grp * ch_per_group

        lane = lax.iota(jnp.int32, LANES)
        neg = jnp.full((LANES,), -jnp.inf, dtype=jnp.float32)

        for p in range(NPASS):
            ch0 = ch_base + p * CPP

            def init_body(j, carry):
                for a in accs:
                    for r in range(LANES):
                        a[r, pl.ds(j * LANES, LANES)] = neg
                return carry
            lax.fori_loop(0, NSEG // LANES, init_body, 0)

            def copies(t, lb_buf, vl_buf, sem):
                off = t * chunk
                return (
                    pltpu.make_async_copy(
                        spx.at[b, pl.ds(off, chunk)], lb_buf, sem),
                    pltpu.make_async_copy(
                        img.at[b, pl.ds(ch0, CPP), pl.ds(off, chunk)],
                        vl_buf, sem),
                )

            def start(t, lb_buf, vl_buf, sem):
                for d in copies(t, lb_buf, vl_buf, sem):
                    d.start()

            def wait(t, lb_buf, vl_buf, sem):
                for d in copies(t, lb_buf, vl_buf, sem):
                    d.wait()

            def compute(lab, val):
                def inner(i, c2):
                    i0 = 2 * i * LANES
                    i1 = i0 + LANES
                    lb_a = lab[pl.ds(i0, LANES)]
                    vs_a = [val[c, pl.ds(i0, LANES)] for c in range(CPP)]
                    curs_a = [plsc.load_gather(accs[c], [lane, lb_a])
                              for c in range(CPP)]
                    news_a = [jnp.maximum(curs_a[c], vs_a[c])
                              for c in range(CPP)]
                    lb_b = lab[pl.ds(i1, LANES)]
                    vs_b = [val[c, pl.ds(i1, LANES)] for c in range(CPP)]
                    for c in range(CPP):
                        plsc.store_scatter(accs[c], [lane, lb_a], news_a[c])
                    curs_b = [plsc.load_gather(accs[c], [lane, lb_b])
                              for c in range(CPP)]
                    news_b = [jnp.maximum(curs_b[c], vs_b[c])
                              for c in range(CPP)]
                    for c in range(CPP):
                        plsc.store_scatter(accs[c], [lane, lb_b], news_b[c])
                    return c2
                lax.fori_loop(0, vregs // 2, inner, 0)

            start(0, lab0, val0, sem0)

            def chunk_body(u, carry):
                t0 = 2 * u
                wait(t0, lab0, val0, sem0)
                start(t0 + 1, lab1, val1, sem1)
                compute(lab0, val0)
                wait(t0 + 1, lab1, val1, sem1)

                @pl.when(u + 1 < nchunks // 2)
                def _():
                    start(t0 + 2, lab0, val0, sem0)
                compute(lab1, val1)
                return carry
            lax.fori_loop(0, nchunks // 2, chunk_body, 0)

            for c in range(CPP):
                a = accs[c]
                for step in (8, 4, 2, 1):
                    def red_body(j, carry, a=a, step=step):
                        for r in range(step):
                            x = a[r, pl.ds(j * LANES, LANES)]
                            y = a[r + step, pl.ds(j * LANES, LANES)]
                            a[r, pl.ds(j * LANES, LANES)] = jnp.maximum(x, y)
                        return carry
                    lax.fori_loop(0, NSEG // LANES, red_body, 0)
                pltpu.sync_copy(a.at[0], out.at[b, ch0 + c])

    run = pl.kernel(
        body,
        out_type=jax.ShapeDtypeStruct((nbatch, nchan, NSEG), jnp.float32),
        mesh=mesh,
        compiler_params=pltpu.CompilerParams(
            use_tc_tiling_on_sc=False, needs_layout_passes=False),
        scratch_types=[
            pltpu.VMEM((chunk,), jnp.int32),
            pltpu.VMEM((CPP, chunk), jnp.float32),
            pltpu.VMEM((chunk,), jnp.int32),
            pltpu.VMEM((CPP, chunk), jnp.float32),
            pltpu.SemaphoreType.DMA,
            pltpu.SemaphoreType.DMA,
        ] + [pltpu.VMEM((LANES, NSEG), jnp.float32)] * CPP,
    )
    return run


def kernel(img, spx):
    B, C, H, W = img.shape
    imgf = img.reshape(B, C, H * W)
    spxf = spx.reshape(B, H * W).astype(jnp.int32)
    run = _build(B, C, H * W, 2048)
    return run(imgf, spxf)
